# baseline (speedup 1.0000x reference)
import jax
import jax.numpy as jnp
from jax.experimental import pallas as pl
from jax.experimental.pallas import tpu as pltpu

_C = 1 << 19
_NBUF = 8


def _shift_body(data_ref, x_ref, out_ref, bufs, lsems, ssems, hsem):
    total = data_ref.shape[0] - 128
    nc = (total + _C - 1) // _C

    def load(k):
        off = k * _C
        sz = min(_C, total - off)
        b = k % _NBUF
        return pltpu.make_async_copy(
            data_ref.at[pl.ds(off, sz)],
            bufs.at[b, pl.ds(0, sz)],
            lsems.at[b],
        )

    def store(k):
        off = k * _C
        sz = min(_C, total - off)
        b = k % _NBUF
        return pltpu.make_async_copy(
            bufs.at[b, pl.ds(0, sz)],
            out_ref.at[pl.ds(128 + off, sz)],
            ssems.at[b],
        )

    loads = [load(k) for k in range(nc)]
    stores = [store(k) for k in range(nc)]

    head = pltpu.make_async_copy(x_ref, out_ref.at[pl.ds(0, 128)], hsem)
    head.start()

    for k in range(min(_NBUF, nc)):
        loads[k].start()
    for k in range(nc):
        loads[k].wait()
        stores[k].start()
        nl = k + 1
        if _NBUF <= nl < nc:
            stores[nl - _NBUF].wait()
            loads[nl].start()
    for k in range(max(0, nc - _NBUF), nc):
        stores[k].wait()
    head.wait()


def kernel(data, x):
    n, d = data.shape
    flat = pl.pallas_call(
        _shift_body,
        in_specs=[
            pl.BlockSpec(memory_space=pl.ANY),
            pl.BlockSpec(memory_space=pl.ANY),
        ],
        out_specs=pl.BlockSpec(memory_space=pl.ANY),
        out_shape=jax.ShapeDtypeStruct((n * d,), data.dtype),
        scratch_shapes=[
            pltpu.VMEM((_NBUF, _C), jnp.float32),
            pltpu.SemaphoreType.DMA((_NBUF,)),
            pltpu.SemaphoreType.DMA((_NBUF,)),
            pltpu.SemaphoreType.DMA,
        ],
    )(data.reshape(-1), x)
    return flat.reshape(n, d)


# speedup vs baseline: 1.0285x; 1.0285x over previous
import functools

import jax
import jax.numpy as jnp
from jax import lax
from jax.experimental import pallas as pl
from jax.experimental.pallas import tpu as pltpu
from jax.experimental.pallas import tpu_sc as plsc

_NW = 32
_WCHUNK = 1 << 19
_C = 1 << 15
_NC = _WCHUNK // _C
_NC0 = _NC - 1
_R0 = _WCHUNK - 128 - _NC0 * _C


def _pipe(data_ref, out_ref, bufs, lsems, ssems, src_base, dst_base, nc):
    def load(k, b):
        return pltpu.make_async_copy(
            data_ref.at[pl.ds(src_base + k * _C, _C)], bufs[b], lsems.at[b]
        )

    def store(k, b):
        return pltpu.make_async_copy(
            bufs[b], out_ref.at[pl.ds(dst_base + k * _C, _C)], ssems.at[b]
        )

    load(0, 0).start()
    for k in range(nc):
        b = k % 2
        load(k, b).wait()
        store(k, b).start()
        if k + 1 < nc:
            if k >= 1:
                store(k - 1, (k - 1) % 2).wait()
            load(k + 1, (k + 1) % 2).start()
    for k in range(max(0, nc - 2), nc):
        store(k, k % 2).wait()


def _sc_body(data_ref, x_ref, out_ref, buf_a, buf_b, rbuf, hbuf,
             lsems, ssems, rsem, hsem):
    c = lax.axis_index("c")
    s = lax.axis_index("s")
    wid = s * 2 + c
    bufs = [buf_a, buf_b]

    @pl.when(wid == 0)
    def _():
        pltpu.make_async_copy(x_ref, hbuf, hsem).start()
        rload = pltpu.make_async_copy(
            data_ref.at[pl.ds(_NC0 * _C, _R0)], rbuf, rsem
        )
        rload.start()
        pltpu.make_async_copy(x_ref, hbuf, hsem).wait()
        hstore = pltpu.make_async_copy(
            hbuf, out_ref.at[pl.ds(0, 128)], hsem
        )
        hstore.start()
        _pipe(data_ref, out_ref, bufs, lsems, ssems, 0, 128, _NC0)
        rload.wait()
        rstore = pltpu.make_async_copy(
            rbuf, out_ref.at[pl.ds(128 + _NC0 * _C, _R0)], rsem
        )
        rstore.start()
        rstore.wait()
        hstore.wait()

    @pl.when(wid != 0)
    def _():
        base = pl.multiple_of(wid * _WCHUNK, 1 << 19)
        _pipe(data_ref, out_ref, bufs, lsems, ssems, base - 128, base, _NC)


def kernel(data, x):
    n, d = data.shape
    mesh = plsc.VectorSubcoreMesh(core_axis_name="c", subcore_axis_name="s")
    sc_fn = functools.partial(
        pl.kernel,
        mesh=mesh,
        out_type=jax.ShapeDtypeStruct((n * d,), data.dtype),
        scratch_types=[
            pltpu.VMEM((_C,), jnp.float32),
            pltpu.VMEM((_C,), jnp.float32),
            pltpu.VMEM((_R0,), jnp.float32),
            pltpu.VMEM((128,), jnp.float32),
            pltpu.SemaphoreType.DMA((2,)),
            pltpu.SemaphoreType.DMA((2,)),
            pltpu.SemaphoreType.DMA,
            pltpu.SemaphoreType.DMA,
        ],
    )(_sc_body)
    flat = sc_fn(data.reshape(-1), x)
    return flat.reshape(n, d)
